# baseline (speedup 1.0000x reference)
import functools

import jax
import jax.numpy as jnp
from jax import lax
from jax.experimental import pallas as pl
from jax.experimental.pallas import tpu as pltpu
from jax.experimental.pallas import tpu_sc as plsc

NUM_CORES = 2
NUM_SUBCORES = 16
LANES = 16
NW = NUM_CORES * NUM_SUBCORES
BATCH = 16384
DIM = 64
BPW = BATCH // NW
CHUNK = 128
NCHUNK = BPW // CHUNK


def _make_kernel():
    mesh = plsc.VectorSubcoreMesh(core_axis_name="c", subcore_axis_name="s")

    @functools.partial(
        pl.kernel,
        mesh=mesh,
        out_type=jax.ShapeDtypeStruct((BATCH,), jnp.float32),
        compiler_params=pltpu.CompilerParams(use_tc_tiling_on_sc=False),
        scratch_types=[
            pltpu.VMEM((NCHUNK, CHUNK), jnp.int32),
            pltpu.VMEM((NCHUNK, CHUNK), jnp.int32),
            pltpu.VMEM((BPW, DIM), jnp.float32),
            pltpu.VMEM((BPW, DIM), jnp.float32),
            pltpu.VMEM((BPW,), jnp.float32),
            pltpu.SemaphoreType.DMA,
        ],
    )
    def lightgcn_dot(uids_hbm, iids_hbm, utab_hbm, itab_hbm, out_hbm,
                     uid_v, iid_v, urows_v, irows_v, out_v, sem):
        wid = lax.axis_index("s") * NUM_CORES + lax.axis_index("c")
        base = wid * BPW

        pltpu.sync_copy(uids_hbm.at[wid], uid_v)
        pltpu.sync_copy(iids_hbm.at[wid], iid_v)

        copies = []
        for j in range(NCHUNK):
            copies.append(pltpu.async_copy(
                utab_hbm.at[uid_v.at[j]],
                urows_v.at[pl.ds(j * CHUNK, CHUNK)], sem))
            copies.append(pltpu.async_copy(
                itab_hbm.at[iid_v.at[j]],
                irows_v.at[pl.ds(j * CHUNK, CHUNK)], sem))
        for c in copies:
            c.wait()

        lane = lax.broadcasted_iota(jnp.int32, (LANES,), 0)
        perms = [lane ^ (1 << k) for k in range(4)]

        dnums = lax.GatherDimensionNumbers(
            offset_dims=(), collapsed_slice_dims=(0,), start_index_map=(0,))

        def _permute(v, idx):
            return lax.gather(
                v, idx[:, None], dnums, slice_sizes=(1,),
                mode=lax.GatherScatterMode.PROMISE_IN_BOUNDS)

        def _allsum(v):
            for p in perms:
                v = v + _permute(v, p)
            return v

        def group_body(g, carry):
            base_row = g * LANES
            res = jnp.zeros((LANES,), jnp.float32)
            for r in range(LANES):
                b = base_row + r
                acc = None
                for c in range(DIM // LANES):
                    u = urows_v[b, pl.ds(c * LANES, LANES)]
                    v = irows_v[b, pl.ds(c * LANES, LANES)]
                    p = u * v
                    acc = p if acc is None else acc + p
                res = jnp.where(lane == r, _allsum(acc), res)
            out_v[pl.ds(base_row, LANES)] = res
            return carry

        lax.fori_loop(0, BPW // LANES, group_body, 0)

        pltpu.sync_copy(out_v, out_hbm.at[pl.ds(base, BPW)])

    return lightgcn_dot


_KERNEL = _make_kernel()


def kernel(user_ids, item_ids, user_table, item_table):
    uids = user_ids.astype(jnp.int32).reshape(NW, NCHUNK, CHUNK)
    iids = item_ids.astype(jnp.int32).reshape(NW, NCHUNK, CHUNK)
    return _KERNEL(uids, iids, user_table, item_table)


# speedup vs baseline: 1.2129x; 1.2129x over previous
import functools

import jax
import jax.numpy as jnp
from jax import lax
from jax.experimental import pallas as pl
from jax.experimental.pallas import tpu as pltpu
from jax.experimental.pallas import tpu_sc as plsc

NUM_CORES = 2
NUM_SUBCORES = 16
LANES = 16
NW = NUM_CORES * NUM_SUBCORES
BATCH = 16384
DIM = 64
NROWS = 1000000
BPW = BATCH // NW
CHUNK = 128
NCHUNK = BPW // CHUNK
PASSES = 2
CPP = NCHUNK // PASSES
EPP = BPW // PASSES
TCW = 1024


def _interleave_body(u_ref, i_ref, out_ref):
    out_ref[:, 0:DIM] = u_ref[...].T
    out_ref[:, DIM:2 * DIM] = i_ref[...].T


def _interleave(ut, it):
    return pl.pallas_call(
        _interleave_body,
        grid=((NROWS + TCW - 1) // TCW,),
        in_specs=[
            pl.BlockSpec((DIM, TCW), lambda i: (0, i)),
            pl.BlockSpec((DIM, TCW), lambda i: (0, i)),
        ],
        out_specs=pl.BlockSpec((TCW, 2 * DIM), lambda i: (i, 0)),
        out_shape=jax.ShapeDtypeStruct((NROWS, 2 * DIM), jnp.float32),
    )(ut, it)


def _make_sc_kernel():
    mesh = plsc.VectorSubcoreMesh(core_axis_name="c", subcore_axis_name="s")

    @functools.partial(
        pl.kernel,
        mesh=mesh,
        out_type=jax.ShapeDtypeStruct((BATCH,), jnp.float32),
        compiler_params=pltpu.CompilerParams(use_tc_tiling_on_sc=False),
        scratch_types=[
            pltpu.VMEM((NCHUNK, CHUNK), jnp.int32),
            pltpu.VMEM((NCHUNK, CHUNK), jnp.int32),
            pltpu.VMEM((EPP, 2 * DIM), jnp.float32),
            pltpu.VMEM((EPP, 2 * DIM), jnp.float32),
            pltpu.VMEM((BPW,), jnp.float32),
            pltpu.SemaphoreType.DMA,
        ],
    )
    def lightgcn_dot(uids_hbm, iids_hbm, cat_hbm, out_hbm,
                     uid_v, iid_v, urows_v, irows_v, out_v, sem):
        wid = lax.axis_index("s") * NUM_CORES + lax.axis_index("c")
        base = wid * BPW

        pltpu.sync_copy(uids_hbm.at[wid], uid_v)
        pltpu.sync_copy(iids_hbm.at[wid], iid_v)

        lane = lax.broadcasted_iota(jnp.int32, (LANES,), 0)
        perms = [lane ^ (1 << k) for k in range(4)]
        dnums = lax.GatherDimensionNumbers(
            offset_dims=(), collapsed_slice_dims=(0,), start_index_map=(0,))

        def _permute(v, idx):
            return lax.gather(
                v, idx[:, None], dnums, slice_sizes=(1,),
                mode=lax.GatherScatterMode.PROMISE_IN_BOUNDS)

        def _allsum(v):
            for p in perms:
                v = v + _permute(v, p)
            return v

        for p in range(PASSES):
            copies = []
            for j in range(CPP):
                copies.append(pltpu.async_copy(
                    cat_hbm.at[uid_v.at[p * CPP + j]],
                    urows_v.at[pl.ds(j * CHUNK, CHUNK)], sem))
                copies.append(pltpu.async_copy(
                    cat_hbm.at[iid_v.at[p * CPP + j]],
                    irows_v.at[pl.ds(j * CHUNK, CHUNK)], sem))
            for c in copies:
                c.wait()

            def group_body(g, carry):
                base_row = g * LANES
                res = jnp.zeros((LANES,), jnp.float32)
                for r in range(LANES):
                    b = base_row + r
                    acc = None
                    for c in range(DIM // LANES):
                        u = urows_v[b, pl.ds(c * LANES, LANES)]
                        v = irows_v[b, pl.ds(DIM + c * LANES, LANES)]
                        prod = u * v
                        acc = prod if acc is None else acc + prod
                    res = jnp.where(lane == r, _allsum(acc), res)
                out_v[pl.ds(p * EPP + base_row, LANES)] = res
                return carry

            lax.fori_loop(0, EPP // LANES, group_body, 0)

        pltpu.sync_copy(out_v, out_hbm.at[pl.ds(base, BPW)])

    return lightgcn_dot


_SC_KERNEL = _make_sc_kernel()


def kernel(user_ids, item_ids, user_table, item_table):
    uids = user_ids.astype(jnp.int32).reshape(NW, NCHUNK, CHUNK)
    iids = item_ids.astype(jnp.int32).reshape(NW, NCHUNK, CHUNK)
    cat = _interleave(user_table.T, item_table.T)
    return _SC_KERNEL(uids, iids, cat)


# speedup vs baseline: 2.0702x; 1.7067x over previous
import functools

import jax
import jax.numpy as jnp
from jax import lax
from jax.experimental import pallas as pl
from jax.experimental.pallas import tpu as pltpu
from jax.experimental.pallas import tpu_sc as plsc

NUM_CORES = 2
NUM_SUBCORES = 16
LANES = 16
NW = NUM_CORES * NUM_SUBCORES
BATCH = 16384
DIM = 64
NROWS = 1000000
BPW = BATCH // NW
CHUNK = 128
NCHUNK = BPW // CHUNK
PASSES = 2
CPP = NCHUNK // PASSES
EPP = BPW // PASSES
TCW = 4096


def _interleave_body(u_ref, i_ref, out_ref):
    out_ref[:, 0:DIM] = u_ref[...].T
    out_ref[:, DIM:2 * DIM] = i_ref[...].T


def _interleave(ut, it):
    return pl.pallas_call(
        _interleave_body,
        grid=((NROWS + TCW - 1) // TCW,),
        in_specs=[
            pl.BlockSpec((DIM, TCW), lambda i: (0, i)),
            pl.BlockSpec((DIM, TCW), lambda i: (0, i)),
        ],
        out_specs=pl.BlockSpec((TCW, 2 * DIM), lambda i: (i, 0)),
        out_shape=jax.ShapeDtypeStruct((NROWS, 2 * DIM), jnp.float32),
    )(ut, it)


def _make_sc_kernel():
    mesh = plsc.VectorSubcoreMesh(core_axis_name="c", subcore_axis_name="s")

    @functools.partial(
        pl.kernel,
        mesh=mesh,
        out_type=jax.ShapeDtypeStruct((BATCH,), jnp.float32),
        compiler_params=pltpu.CompilerParams(use_tc_tiling_on_sc=False),
        scratch_types=[
            pltpu.VMEM((NCHUNK, CHUNK), jnp.int32),
            pltpu.VMEM((NCHUNK, CHUNK), jnp.int32),
            pltpu.VMEM((EPP, 2 * DIM), jnp.float32),
            pltpu.VMEM((EPP, 2 * DIM), jnp.float32),
            pltpu.VMEM((BPW,), jnp.float32),
            pltpu.SemaphoreType.DMA,
        ],
    )
    def lightgcn_dot(uids_hbm, iids_hbm, cat_hbm, out_hbm,
                     uid_v, iid_v, urows_v, irows_v, out_v, sem):
        wid = lax.axis_index("s") * NUM_CORES + lax.axis_index("c")
        base = wid * BPW

        pltpu.sync_copy(uids_hbm.at[wid], uid_v)
        pltpu.sync_copy(iids_hbm.at[wid], iid_v)

        lane = lax.broadcasted_iota(jnp.int32, (LANES,), 0)
        perms = [lane ^ (1 << k) for k in range(4)]
        dnums = lax.GatherDimensionNumbers(
            offset_dims=(), collapsed_slice_dims=(0,), start_index_map=(0,))

        def _permute(v, idx):
            return lax.gather(
                v, idx[:, None], dnums, slice_sizes=(1,),
                mode=lax.GatherScatterMode.PROMISE_IN_BOUNDS)

        def _allsum(v):
            for p in perms:
                v = v + _permute(v, p)
            return v

        for p in range(PASSES):
            copies = []
            for j in range(CPP):
                copies.append(pltpu.async_copy(
                    cat_hbm.at[uid_v.at[p * CPP + j]],
                    urows_v.at[pl.ds(j * CHUNK, CHUNK)], sem))
                copies.append(pltpu.async_copy(
                    cat_hbm.at[iid_v.at[p * CPP + j]],
                    irows_v.at[pl.ds(j * CHUNK, CHUNK)], sem))
            for c in copies:
                c.wait()

            def group_body(g, carry):
                base_row = g * LANES
                res = jnp.zeros((LANES,), jnp.float32)
                for r in range(LANES):
                    b = base_row + r
                    acc = None
                    for c in range(DIM // LANES):
                        u = urows_v[b, pl.ds(c * LANES, LANES)]
                        v = irows_v[b, pl.ds(DIM + c * LANES, LANES)]
                        prod = u * v
                        acc = prod if acc is None else acc + prod
                    res = jnp.where(lane == r, _allsum(acc), res)
                out_v[pl.ds(p * EPP + base_row, LANES)] = res
                return carry

            lax.fori_loop(0, EPP // LANES, group_body, 0)

        pltpu.sync_copy(out_v, out_hbm.at[pl.ds(base, BPW)])

    return lightgcn_dot


_SC_KERNEL = _make_sc_kernel()


def kernel(user_ids, item_ids, user_table, item_table):
    uids = user_ids.astype(jnp.int32).reshape(NW, NCHUNK, CHUNK)
    iids = item_ids.astype(jnp.int32).reshape(NW, NCHUNK, CHUNK)
    cat = _interleave(user_table.T, item_table.T)
    return _SC_KERNEL(uids, iids, cat)


# speedup vs baseline: 2.3765x; 1.1480x over previous
import functools

import jax
import jax.numpy as jnp
from jax import lax
from jax.experimental import pallas as pl
from jax.experimental.pallas import tpu as pltpu
from jax.experimental.pallas import tpu_sc as plsc

NUM_CORES = 2
NUM_SUBCORES = 16
LANES = 16
NW = NUM_CORES * NUM_SUBCORES
BATCH = 16384
DIM = 64
NROWS = 1000000
BPW = BATCH // NW
CHUNK = 128
NCHUNK = BPW // CHUNK
PASSES = 2
CPP = NCHUNK // PASSES
EPP = BPW // PASSES
TCW = 8192


def _interleave_body(u_ref, i_ref, out_ref):
    out_ref[:, 0:DIM] = u_ref[...].T
    out_ref[:, DIM:2 * DIM] = i_ref[...].T


def _interleave(ut, it):
    return pl.pallas_call(
        _interleave_body,
        grid=((NROWS + TCW - 1) // TCW,),
        in_specs=[
            pl.BlockSpec((DIM, TCW), lambda i: (0, i)),
            pl.BlockSpec((DIM, TCW), lambda i: (0, i)),
        ],
        out_specs=pl.BlockSpec((TCW, 2 * DIM), lambda i: (i, 0)),
        out_shape=jax.ShapeDtypeStruct((NROWS, 2 * DIM), jnp.float32),
    )(ut, it)


def _make_sc_kernel():
    mesh = plsc.VectorSubcoreMesh(core_axis_name="c", subcore_axis_name="s")

    @functools.partial(
        pl.kernel,
        mesh=mesh,
        out_type=jax.ShapeDtypeStruct((BATCH,), jnp.float32),
        compiler_params=pltpu.CompilerParams(use_tc_tiling_on_sc=False),
        scratch_types=[
            pltpu.VMEM((NCHUNK, CHUNK), jnp.int32),
            pltpu.VMEM((NCHUNK, CHUNK), jnp.int32),
            pltpu.VMEM((EPP, 2 * DIM), jnp.float32),
            pltpu.VMEM((EPP, 2 * DIM), jnp.float32),
            pltpu.VMEM((BPW,), jnp.float32),
            pltpu.SemaphoreType.DMA,
        ],
    )
    def lightgcn_dot(uids_hbm, iids_hbm, cat_hbm, out_hbm,
                     uid_v, iid_v, urows_v, irows_v, out_v, sem):
        wid = lax.axis_index("s") * NUM_CORES + lax.axis_index("c")
        base = wid * BPW

        pltpu.sync_copy(uids_hbm.at[wid], uid_v)
        pltpu.sync_copy(iids_hbm.at[wid], iid_v)

        lane = lax.broadcasted_iota(jnp.int32, (LANES,), 0)
        perms = [lane ^ (1 << k) for k in range(4)]
        dnums = lax.GatherDimensionNumbers(
            offset_dims=(), collapsed_slice_dims=(0,), start_index_map=(0,))

        def _permute(v, idx):
            return lax.gather(
                v, idx[:, None], dnums, slice_sizes=(1,),
                mode=lax.GatherScatterMode.PROMISE_IN_BOUNDS)

        def _allsum(v):
            for p in perms:
                v = v + _permute(v, p)
            return v

        for p in range(PASSES):
            copies = []
            for j in range(CPP):
                copies.append(pltpu.async_copy(
                    cat_hbm.at[uid_v.at[p * CPP + j]],
                    urows_v.at[pl.ds(j * CHUNK, CHUNK)], sem))
                copies.append(pltpu.async_copy(
                    cat_hbm.at[iid_v.at[p * CPP + j]],
                    irows_v.at[pl.ds(j * CHUNK, CHUNK)], sem))
            for c in copies:
                c.wait()

            def group_body(g, carry):
                base_row = g * LANES
                res = jnp.zeros((LANES,), jnp.float32)
                for r in range(LANES):
                    b = base_row + r
                    acc = None
                    for c in range(DIM // LANES):
                        u = urows_v[b, pl.ds(c * LANES, LANES)]
                        v = irows_v[b, pl.ds(DIM + c * LANES, LANES)]
                        prod = u * v
                        acc = prod if acc is None else acc + prod
                    res = jnp.where(lane == r, _allsum(acc), res)
                out_v[pl.ds(p * EPP + base_row, LANES)] = res
                return carry

            lax.fori_loop(0, EPP // LANES, group_body, 0)

        pltpu.sync_copy(out_v, out_hbm.at[pl.ds(base, BPW)])

    return lightgcn_dot


_SC_KERNEL = _make_sc_kernel()


def kernel(user_ids, item_ids, user_table, item_table):
    uids = user_ids.astype(jnp.int32).reshape(NW, NCHUNK, CHUNK)
    iids = item_ids.astype(jnp.int32).reshape(NW, NCHUNK, CHUNK)
    cat = _interleave(user_table.T, item_table.T)
    return _SC_KERNEL(uids, iids, cat)
